# Initial kernel scaffold; baseline (speedup 1.0000x reference)
import jax
import jax.numpy as jnp
from jax.experimental import pallas as pl


def kernel(hidden_states, gate_w, W1, b1, W2, b2):
    raise NotImplementedError("write your pallas kernel here")


# speedup vs baseline: 3.0137x; 3.0137x over previous
import functools

import jax
import jax.numpy as jnp
from jax import lax
from jax.experimental import pallas as pl
from jax.experimental.pallas import tpu as pltpu
from jax.experimental.pallas import tpu_sc as plsc

E = 8
TOP_K = 2
D = 1024
F = 2048
T = 2048
P = T * TOP_K
TILE = 256
NP = 6144
NT = 23
_MIN_PROB = 0.001


def _router_body(x_ref, gw_ref, w0_ref, w1_ref, d0_ref, d1_ref, te_ref, act_ref):
    x = x_ref[...]
    gw = gw_ref[...]
    logits = lax.dot_general(x, gw, (((1,), (1,)), ((), ())),
                             preferred_element_type=jnp.float32)
    m = jnp.max(logits, axis=1, keepdims=True)
    un = jnp.exp(logits - m)
    sc = un / jnp.sum(un, axis=1, keepdims=True)
    sc = sc * (1.0 - _MIN_PROB * E) + _MIN_PROB
    sc = sc / jnp.sum(sc, axis=1, keepdims=True)

    ecol = lax.broadcasted_iota(jnp.int32, (T, E), 1)
    v0 = jnp.max(sc, axis=1, keepdims=True)
    i0 = jnp.min(jnp.where(sc == v0, ecol, E), axis=1, keepdims=True)
    scm = jnp.where(ecol == i0, -jnp.inf, sc)
    v1 = jnp.max(scm, axis=1, keepdims=True)
    i1 = jnp.min(jnp.where(scm == v1, ecol, E), axis=1, keepdims=True)
    wsum = v0 + v1
    w0_ref[...] = v0 / wsum
    w1_ref[...] = v1 / wsum

    oh0 = (ecol == i0).astype(jnp.float32)
    oh1 = (ecol == i1).astype(jnp.float32)
    ri = lax.broadcasted_iota(jnp.int32, (128, 128), 0)
    ci = lax.broadcasted_iota(jnp.int32, (128, 128), 1)
    ls = (ri > ci).astype(jnp.float32)

    def _ranks(oh, carry):
        parts = []
        for c in range(T // 128):
            blk = oh[c * 128:(c + 1) * 128, :]
            pre = lax.dot_general(ls, blk, (((1,), (0,)), ((), ())),
                                  preferred_element_type=jnp.float32) + carry
            parts.append(jnp.sum(pre * blk, axis=1, keepdims=True))
            carry = carry + jnp.sum(blk, axis=0, keepdims=True)
        return jnp.concatenate(parts, axis=0), carry

    zero8 = jnp.zeros((1, E), jnp.float32)
    rank0, carry = _ranks(oh0, zero8)
    rank1, counts = _ranks(oh1, carry)

    cnt = counts.astype(jnp.int32)
    pc = ((cnt + (TILE - 1)) // TILE) * TILE
    r8 = lax.broadcasted_iota(jnp.int32, (E, E), 0)
    c8 = lax.broadcasted_iota(jnp.int32, (E, E), 1)
    m8 = (r8 < c8).astype(jnp.float32)
    off = lax.dot_general(pc.astype(jnp.float32), m8, (((1,), (0,)), ((), ())),
                          preferred_element_type=jnp.float32)
    off_tok0 = jnp.sum(oh0 * off, axis=1, keepdims=True)
    off_tok1 = jnp.sum(oh1 * off, axis=1, keepdims=True)
    d0_ref[...] = (off_tok0 + rank0).astype(jnp.int32)
    d1_ref[...] = (off_tok1 + rank1).astype(jnp.int32)

    starts = lax.broadcasted_iota(jnp.int32, (32, E), 0) * TILE
    offi = off.astype(jnp.int32)
    inb = (starts >= offi) & (starts < offi + pc)
    e32 = lax.broadcasted_iota(jnp.int32, (32, E), 1)
    te = jnp.sum(jnp.where(inb, e32 + 1, 0), axis=1, keepdims=True) - 1
    te_ref[...] = te
    act_ref[...] = (te >= 0).astype(jnp.int32)


def _run_router(x, gate_w):
    outs = (
        jax.ShapeDtypeStruct((T, 1), jnp.float32),
        jax.ShapeDtypeStruct((T, 1), jnp.float32),
        jax.ShapeDtypeStruct((T, 1), jnp.int32),
        jax.ShapeDtypeStruct((T, 1), jnp.int32),
        jax.ShapeDtypeStruct((32, 1), jnp.int32),
        jax.ShapeDtypeStruct((32, 1), jnp.int32),
    )
    return pl.pallas_call(_router_body, out_shape=outs)(x, gate_w)


def _experts_body(te_ref, act_ref, xs_ref, w1_ref, b1_ref, w2_ref, b2_ref, ys_ref):
    j = pl.program_id(0)

    @pl.when(act_ref[j] == 1)
    def _():
        xt = xs_ref[...]
        h = jnp.dot(xt, w1_ref[0], preferred_element_type=jnp.float32)
        h = h + b1_ref[0]
        h = 0.5 * h * (lax.erf(h / 2.0**0.5) + 1.0)
        y = jnp.dot(h, w2_ref[0], preferred_element_type=jnp.float32)
        ys_ref[...] = y + b2_ref[0]


def _run_experts(te_c, act, xs, W1, b1, W2, b2):
    grid_spec = pltpu.PrefetchScalarGridSpec(
        num_scalar_prefetch=2,
        grid=(NT,),
        in_specs=[
            pl.BlockSpec((TILE, D), lambda j, te, act: (j, 0)),
            pl.BlockSpec((1, D, F), lambda j, te, act: (te[j], 0, 0)),
            pl.BlockSpec((1, 1, F), lambda j, te, act: (te[j], 0, 0)),
            pl.BlockSpec((1, F, D), lambda j, te, act: (te[j], 0, 0)),
            pl.BlockSpec((1, 1, D), lambda j, te, act: (te[j], 0, 0)),
        ],
        out_specs=pl.BlockSpec((TILE, D), lambda j, te, act: (j, 0)),
    )
    return pl.pallas_call(
        _experts_body,
        grid_spec=grid_spec,
        out_shape=jax.ShapeDtypeStruct((NP, D), jnp.float32),
    )(te_c, act, xs, W1, b1.reshape(E, 1, F), W2, b2.reshape(E, 1, D))


def _make_dispatch():
    mesh = plsc.VectorSubcoreMesh(core_axis_name="c", subcore_axis_name="s")

    @functools.partial(
        pl.kernel,
        mesh=mesh,
        out_type=jax.ShapeDtypeStruct((NP, D), jnp.float32),
        scratch_types=[
            pltpu.VMEM((64,), jnp.int32),
            pltpu.VMEM((64, D), jnp.float32),
            pltpu.SemaphoreType.DMA,
        ],
    )
    def dispatch(x_hbm, dst_hbm, xs_hbm, didx_v, rows_v, sem):
        wid = lax.axis_index("s") * 2 + lax.axis_index("c")
        base = wid * (P // 32)
        tb = lax.rem(base, T)
        for c in range(2):
            pltpu.sync_copy(x_hbm.at[pl.ds(tb + c * 64, 64)], rows_v)
            pltpu.sync_copy(dst_hbm.at[pl.ds(base + c * 64, 64)], didx_v)
            pltpu.async_copy(rows_v, xs_hbm.at[didx_v], sem).wait()

    return dispatch


def _make_combine():
    mesh = plsc.VectorSubcoreMesh(core_axis_name="c", subcore_axis_name="s")

    @functools.partial(
        pl.kernel,
        mesh=mesh,
        out_type=jax.ShapeDtypeStruct((T, D), jnp.float32),
        scratch_types=[
            pltpu.VMEM((16,), jnp.int32),
            pltpu.VMEM((16,), jnp.int32),
            pltpu.VMEM((16, D), jnp.float32),
            pltpu.VMEM((16, D), jnp.float32),
            pltpu.VMEM((16, 16), jnp.float32),
            pltpu.VMEM((16, 16), jnp.float32),
            pltpu.VMEM((16, D), jnp.float32),
            pltpu.SemaphoreType.DMA,
            pltpu.SemaphoreType.DMA,
        ],
    )
    def combine(ys_hbm, dst_hbm, w0e_hbm, w1e_hbm, out_hbm,
                d0_v, d1_v, r0_v, r1_v, w0m, w1m, o_v, sem0, sem1):
        wid = lax.axis_index("s") * 2 + lax.axis_index("c")
        for c in range(4):
            tb = wid * (T // 32) + c * 16
            pltpu.sync_copy(dst_hbm.at[pl.ds(tb, 16)], d0_v)
            pltpu.sync_copy(dst_hbm.at[pl.ds(T + tb, 16)], d1_v)
            cp0 = pltpu.async_copy(ys_hbm.at[d0_v], r0_v, sem0)
            cp1 = pltpu.async_copy(ys_hbm.at[d1_v], r1_v, sem1)
            pltpu.sync_copy(w0e_hbm.at[pl.ds(tb, 16)], w0m)
            pltpu.sync_copy(w1e_hbm.at[pl.ds(tb, 16)], w1m)
            cp0.wait()
            cp1.wait()
            for t in range(16):
                w0 = w0m[t, :]
                w1 = w1m[t, :]

                def body(g, carry, t=t, w0=w0, w1=w1):
                    ds = pl.ds(g * 16, 16)
                    o_v[t, ds] = w0 * r0_v[t, ds] + w1 * r1_v[t, ds]
                    return carry

                lax.fori_loop(0, D // 16, body, 0)
            pltpu.sync_copy(o_v, out_hbm.at[pl.ds(tb, 16)])

    return combine


def kernel(hidden_states, gate_w, W1, b1, W2, b2):
    Bc, Sc, Dc = hidden_states.shape
    x = hidden_states.reshape(T, D)
    w0, w1, d0, d1, te, act = _run_router(x, gate_w)

    dst = jnp.concatenate([d0[:, 0], d1[:, 0]])
    te23 = te[:NT, 0]
    act23 = act[:NT, 0]
    te_c = jnp.where(act23 == 1, te23, jnp.max(te23))

    xs = _make_dispatch()(x, dst)
    ys = _run_experts(te_c, act23, xs, W1, b1, W2, b2)
    w0e = jnp.broadcast_to(w0, (T, 16))
    w1e = jnp.broadcast_to(w1, (T, 16))
    out = _make_combine()(ys, dst, w0e, w1e)
    return out.reshape(Bc, Sc, Dc)
